# Initial kernel scaffold; baseline (speedup 1.0000x reference)
import jax
import jax.numpy as jnp
from jax.experimental import pallas as pl


def kernel(x, edge_index, att_l, att_r, W1, b1, W2, b2):
    raise NotImplementedError("write your pallas kernel here")


# speedup vs baseline: 20.3161x; 20.3161x over previous
import functools

import jax
import jax.numpy as jnp
from jax import lax
from jax.experimental import pallas as pl
from jax.experimental.pallas import tpu as pltpu
from jax.experimental.pallas import tpu_sc as plsc

EPS = 0.1
NC, NS, L = 2, 16, 16
NW = NC * NS
CH = 128


def _sc_mesh():
    return plsc.VectorSubcoreMesh(core_axis_name="c", subcore_axis_name="s",
                                  num_cores=NC, num_subcores=NS)


def _make_deg(NT, NCH):
    EPT = NCH * CH
    SPT = NT // NS

    @functools.partial(
        pl.kernel,
        out_type=jax.ShapeDtypeStruct((NC * NT,), jnp.float32),
        mesh=_sc_mesh(),
        scratch_types=[pltpu.VMEM((SPT,), jnp.float32),
                       pltpu.VMEM((CH,), jnp.int32),
                       pltpu.VMEM((CH,), jnp.float32),
                       pltpu.VMEM_SHARED((NT,), jnp.float32)],
    )
    def deg_kernel(col_hbm, out_hbm, zeros_v, idxc, ones_v, sdeg):
        cid = lax.axis_index("c")
        sid = lax.axis_index("s")
        wid = cid * NS + sid

        zero = jnp.zeros((L,), jnp.float32)
        one = jnp.ones((L,), jnp.float32)

        def zbody(i, c):
            zeros_v[pl.ds(i * L, L)] = zero
            return c
        lax.fori_loop(0, SPT // L, zbody, 0)
        for g in range(CH // L):
            ones_v[pl.ds(g * L, L)] = one

        pltpu.sync_copy(zeros_v, sdeg.at[pl.ds(sid * SPT, SPT)])
        plsc.subcore_barrier()

        def cbody(k, c):
            base = wid * EPT + k * CH
            pltpu.sync_copy(col_hbm.at[pl.ds(base, CH)], idxc)
            pltpu.sync_copy(ones_v, sdeg.at[idxc], add=True)
            return c
        lax.fori_loop(0, NCH, cbody, 0)
        plsc.subcore_barrier()

        pltpu.sync_copy(sdeg.at[pl.ds(sid * SPT, SPT)],
                        out_hbm.at[pl.ds(cid * NT + sid * SPT, SPT)])

    return deg_kernel


def _make_stats(N, D, BLK):
    def body(x_ref, degt_ref, attl_ref, attr_ref,
             al_ref, ar_ref, dis_ref, cs_ref):
        xb = x_ref[...]
        al = jnp.sum(xb * attl_ref[...], axis=1, keepdims=True)
        ar = jnp.sum(xb * attr_ref[...], axis=1, keepdims=True)
        tot = jnp.sum(degt_ref[...], axis=1, keepdims=True) + 1.0
        al_ref[...] = al
        ar_ref[...] = ar
        dis_ref[...] = lax.rsqrt(tot)
        cs_ref[...] = jnp.tanh(al + ar) / tot + EPS

    return pl.pallas_call(
        body,
        grid=(N // BLK,),
        in_specs=[pl.BlockSpec((BLK, D), lambda i: (i, 0)),
                  pl.BlockSpec((BLK, NC), lambda i: (i, 0)),
                  pl.BlockSpec((1, D), lambda i: (0, 0)),
                  pl.BlockSpec((1, D), lambda i: (0, 0))],
        out_specs=[pl.BlockSpec((BLK, 1), lambda i: (i, 0))] * 4,
        out_shape=[jax.ShapeDtypeStruct((N, 1), jnp.float32)] * 4,
    )


def _make_edge(N, D, NT, NCH):
    EPT = NCH * CH
    RPT = NT // NS

    @functools.partial(
        pl.kernel,
        out_type=jax.ShapeDtypeStruct((NC * NT, D), jnp.float32),
        mesh=_sc_mesh(),
        compiler_params=pltpu.CompilerParams(needs_layout_passes=False),
        scratch_types=[
            pltpu.VMEM((NT,), jnp.float32),
            pltpu.VMEM((NT,), jnp.float32),
            pltpu.VMEM((NT,), jnp.float32),
            pltpu.VMEM((CH,), jnp.int32),
            pltpu.VMEM((CH,), jnp.int32),
            pltpu.VMEM((CH, D), jnp.float32),
            pltpu.VMEM((CH,), jnp.float32),
            pltpu.VMEM((8, D), jnp.float32),
            pltpu.VMEM_SHARED((NT, D), jnp.float32),
            pltpu.SemaphoreType.DMA,
        ],
    )
    def edge_kernel(row_hbm, col_hbm, x_hbm, al_hbm, ar_hbm, dis_hbm,
                    out_hbm, al_v, ar_v, dis_v, idxr, idxc, rows, cvals,
                    zbuf, agg, sem):
        cid = lax.axis_index("c")
        sid = lax.axis_index("s")
        wid = cid * NS + sid

        pltpu.sync_copy(al_hbm, al_v)
        pltpu.sync_copy(ar_hbm, ar_v)
        pltpu.sync_copy(dis_hbm, dis_v)

        zero = jnp.zeros((L,), jnp.float32)
        for r in range(8):
            for j in range(D // L):
                zbuf[r, pl.ds(j * L, L)] = zero

        def za(i, c):
            pltpu.sync_copy(zbuf, agg.at[pl.ds(sid * RPT + i * 8, 8)])
            return c
        lax.fori_loop(0, RPT // 8, za, 0)
        plsc.subcore_barrier()

        def cbody(k, c):
            base = wid * EPT + k * CH
            pltpu.sync_copy(row_hbm.at[pl.ds(base, CH)], idxr)
            pltpu.sync_copy(col_hbm.at[pl.ds(base, CH)], idxc)
            pltpu.async_copy(x_hbm.at[idxr], rows, sem).wait()
            for g in range(CH // L):
                rg = idxr[pl.ds(g * L, L)]
                cg = idxc[pl.ds(g * L, L)]
                av = plsc.load_gather(al_v, [rg])
                bv = plsc.load_gather(ar_v, [cg])
                dr = plsc.load_gather(dis_v, [rg])
                dc = plsc.load_gather(dis_v, [cg])
                z2 = 2.0 * (av + bv)
                t = 1.0 - 2.0 / (jnp.exp(z2) + 1.0)
                cvals[pl.ds(g * L, L)] = t * dr * dc

            def sbody(e, c2):
                se = plsc.load_gather(cvals, [jnp.full((L,), e, jnp.int32)])
                for j in range(D // L):
                    sl = pl.ds(j * L, L)
                    rows[e, sl] = rows[e, sl] * se
                return c2
            lax.fori_loop(0, CH, sbody, 0)
            pltpu.sync_copy(rows, agg.at[idxc], add=True)
            return c
        lax.fori_loop(0, NCH, cbody, 0)
        plsc.subcore_barrier()

        pltpu.sync_copy(agg.at[pl.ds(sid * RPT, RPT)],
                        out_hbm.at[pl.ds(cid * NT + sid * RPT, RPT)])

    return edge_kernel


def _make_mlp(N, D, BLK):
    def body(a0_ref, a1_ref, x_ref, cs_ref, w1_ref, b1_ref, w2_ref, b2_ref,
             o_ref):
        outb = a0_ref[...] + a1_ref[...] + cs_ref[...] * x_ref[...]
        h = lax.dot_general(outb, w1_ref[...], (((1,), (1,)), ((), ())),
                            preferred_element_type=jnp.float32) + b1_ref[...]
        h = jnp.where(h > 0, h, jnp.exp(jnp.minimum(h, 0.0)) - 1.0)
        o_ref[...] = lax.dot_general(h, w2_ref[...], (((1,), (1,)), ((), ())),
                                     preferred_element_type=jnp.float32) \
            + b2_ref[...]

    full = lambda i: (0, 0)
    blk = lambda i: (i, 0)
    return pl.pallas_call(
        body,
        grid=(N // BLK,),
        in_specs=[pl.BlockSpec((BLK, D), blk),
                  pl.BlockSpec((BLK, D), blk),
                  pl.BlockSpec((BLK, D), blk),
                  pl.BlockSpec((BLK, 1), blk),
                  pl.BlockSpec((D, D), full),
                  pl.BlockSpec((1, D), full),
                  pl.BlockSpec((D, D), full),
                  pl.BlockSpec((1, D), full)],
        out_specs=pl.BlockSpec((BLK, D), blk),
        out_shape=jax.ShapeDtypeStruct((N, D), jnp.float32),
    )


def kernel(x, edge_index, att_l, att_r, W1, b1, W2, b2):
    N, D = x.shape
    E = edge_index.shape[1]
    EPC = NW * CH
    NCH = -(-E // EPC)
    EP = NCH * EPC
    NT = ((N + 1 + 255) // 256) * 256
    BLK = 1000

    pad = EP - E
    rowp = jnp.concatenate([edge_index[0],
                            jnp.zeros((pad,), edge_index.dtype)])
    colp = jnp.concatenate([edge_index[1],
                            jnp.full((pad,), N, edge_index.dtype)])

    degp = _make_deg(NT, NCH)(colp)
    degt = jnp.stack([degp[:N], degp[NT:NT + N]], axis=1)
    al, ar, dis, cs = _make_stats(N, D, BLK)(x, degt, att_l, att_r)

    zpad = jnp.zeros((NT - N,), jnp.float32)
    al_t = jnp.concatenate([al[:, 0], zpad])
    ar_t = jnp.concatenate([ar[:, 0], zpad])
    dis_t = jnp.concatenate([dis[:, 0], zpad])

    aggp = _make_edge(N, D, NT, NCH)(rowp, colp, x, al_t, ar_t, dis_t)
    return _make_mlp(N, D, BLK)(aggp[:N], aggp[NT:NT + N], x, cs,
                                W1, b1.reshape(1, D), W2, b2.reshape(1, D))
